# Initial kernel scaffold; baseline (speedup 1.0000x reference)
import jax
import jax.numpy as jnp
from jax.experimental import pallas as pl


def kernel(x, edge_index, edge_attr, We0, be0, We1, be1, W00, b00, W01, b01, W10, b10, W11, b11, g0, bb0, g1, bb1, Wm0, bm0, Wm1, bm1):
    raise NotImplementedError("write your pallas kernel here")


# speedup vs baseline: 3.0291x; 3.0291x over previous
import functools

import jax
import jax.numpy as jnp
from jax import lax
from jax.experimental import pallas as pl
from jax.experimental.pallas import tpu as pltpu
from jax.experimental.pallas import tpu_sc as plsc

N = 10000
E = 320000
DN = 128
DE = 16
NPAD = 10240
NW = 32
EPW = E // NW
CHUNK = 80
NCHUNK = EPW // CHUNK


_EBLK = 3200


def _edge_embed_body(ea_ref, w_ref, b_ref, e_ref):
    e_ref[...] = (
        jnp.dot(ea_ref[...], w_ref[...], preferred_element_type=jnp.float32)
        + b_ref[...]
    )


def _edge_embed(edge_attr, W, b):
    return pl.pallas_call(
        _edge_embed_body,
        out_shape=jax.ShapeDtypeStruct((E, DN), jnp.float32),
        grid=(E // _EBLK,),
        in_specs=[
            pl.BlockSpec((_EBLK, DE), lambda i: (i, 0)),
            pl.BlockSpec((DE, DN), lambda i: (0, 0)),
            pl.BlockSpec((1, DN), lambda i: (0, 0)),
        ],
        out_specs=pl.BlockSpec((_EBLK, DN), lambda i: (i, 0)),
    )(edge_attr, W, b.reshape(1, DN))


_SC_MESH = plsc.VectorSubcoreMesh(core_axis_name="c", subcore_axis_name="s")


def _sc_aggregate(x, src, dst, e):
    @functools.partial(
        pl.kernel,
        out_type=jax.ShapeDtypeStruct((2, NPAD, DN), jnp.float32),
        mesh=_SC_MESH,
        scratch_types=[
            pltpu.VMEM((CHUNK,), jnp.int32),
            pltpu.VMEM((CHUNK,), jnp.int32),
            pltpu.VMEM((CHUNK, DN), jnp.float32),
            pltpu.VMEM((CHUNK, DN), jnp.float32),
            pltpu.VMEM((128, DN), jnp.float32),
            pltpu.VMEM_SHARED((NPAD, DN), jnp.float32),
            pltpu.SemaphoreType.DMA,
        ],
    )
    def agg(x_hbm, src_hbm, dst_hbm, e_hbm, out_hbm, srcv, dstv, xv, ev, zv, acc, sem):
        cid = lax.axis_index("c")
        sid = lax.axis_index("s")

        @pl.loop(0, 128)
        def _zfill(r):
            for j in range(8):
                zv[r, pl.ds(j * 16, 16)] = jnp.zeros((16,), jnp.float32)

        @pl.loop(0, 5)
        def _zcopy(k):
            pltpu.sync_copy(zv, acc.at[pl.ds(sid * 640 + k * 128, 128)])

        plsc.subcore_barrier()

        wid = cid * 16 + sid
        base = wid * EPW

        @pl.loop(0, NCHUNK)
        def _edge_chunk(c):
            off = base + c * CHUNK
            pltpu.sync_copy(src_hbm.at[pl.ds(off, CHUNK)], srcv)
            gather = pltpu.async_copy(x_hbm.at[srcv], xv, sem)
            pltpu.sync_copy(dst_hbm.at[pl.ds(off, CHUNK)], dstv)
            pltpu.sync_copy(e_hbm.at[pl.ds(off, CHUNK)], ev)
            gather.wait()

            @pl.loop(0, CHUNK)
            def _row(r):
                for j in range(8):
                    sl = (r, pl.ds(j * 16, 16))
                    xv[sl] = jnp.maximum(xv[sl] + ev[sl], 0.0)

            pltpu.sync_copy(xv, acc.at[dstv], add=True)

        plsc.subcore_barrier()
        pltpu.sync_copy(
            acc.at[pl.ds(sid * 640, 640)],
            out_hbm.at[cid, pl.ds(sid * 640, 640)],
        )

    return agg(x, src, dst, e)


def _node_update_body(x_ref, p_ref, w1_ref, b1_ref, w2_ref, b2_ref, g_ref,
                      bb_ref, o_ref):
    h = x_ref[...] + p_ref[0, :N, :] + p_ref[1, :N, :]
    t = jnp.dot(h, w1_ref[...], preferred_element_type=jnp.float32) + b1_ref[...]
    t = jnp.maximum(t, 0.0)
    h2 = jnp.dot(t, w2_ref[...], preferred_element_type=jnp.float32) + b2_ref[...]
    mu = jnp.mean(h2, axis=0, keepdims=True)
    var = jnp.mean(jnp.square(h2 - mu), axis=0, keepdims=True)
    hb = g_ref[...] * (h2 - mu) * lax.rsqrt(var + 1e-5) + bb_ref[...]
    o_ref[...] = jnp.maximum(hb, 0.0)


def _node_update(x, parts, W1, b1, W2, b2, g, bb):
    return pl.pallas_call(
        _node_update_body,
        out_shape=jax.ShapeDtypeStruct((N, DN), jnp.float32),
    )(x, parts, W1, b1.reshape(1, -1), W2, b2.reshape(1, -1),
      g.reshape(1, -1), bb.reshape(1, -1))


def _final_mlp_body(h_ref, w0_ref, b0_ref, w1_ref, b1_ref, o_ref):
    t = jnp.dot(h_ref[...], w0_ref[...], preferred_element_type=jnp.float32)
    t = jnp.maximum(t + b0_ref[...], 0.0)
    o_ref[...] = (
        jnp.dot(t, w1_ref[...], preferred_element_type=jnp.float32) + b1_ref[...]
    )


def _final_mlp(h, Wm0, bm0, Wm1, bm1):
    return pl.pallas_call(
        _final_mlp_body,
        out_shape=jax.ShapeDtypeStruct((N, 1), jnp.float32),
    )(h, Wm0, bm0.reshape(1, -1), Wm1, bm1.reshape(1, -1))


def kernel(x, edge_index, edge_attr, We0, be0, We1, be1, W00, b00, W01, b01,
           W10, b10, W11, b11, g0, bb0, g1, bb1, Wm0, bm0, Wm1, bm1):
    src = edge_index[0]
    dst = edge_index[1]

    e0 = _edge_embed(edge_attr, We0, be0)
    e1 = _edge_embed(edge_attr, We1, be1)

    p0 = _sc_aggregate(x, src, dst, e0)
    h = _node_update(x, p0, W00, b00, W01, b01, g0, bb0)

    p1 = _sc_aggregate(h, src, dst, e1)
    h = _node_update(h, p1, W10, b10, W11, b11, g1, bb1)

    return _final_mlp(h, Wm0, bm0, Wm1, bm1)


# speedup vs baseline: 3.8530x; 1.2720x over previous
import functools

import jax
import jax.numpy as jnp
from jax import lax
from jax.experimental import pallas as pl
from jax.experimental.pallas import tpu as pltpu
from jax.experimental.pallas import tpu_sc as plsc

N = 10000
E = 320000
DN = 128
DE = 16
NPAD = 10240
NW = 32
EPW = E // NW
CHUNK = 40
NCHUNK = EPW // CHUNK
IB = 32
NIBPAD = 256


_EBLK = 3200


def _edge_embed_body(ea_ref, w_ref, b_ref, e_ref):
    e_ref[...] = (
        jnp.dot(ea_ref[...], w_ref[...], preferred_element_type=jnp.float32)
        + b_ref[...]
    )


def _edge_embed(edge_attr, W, b):
    return pl.pallas_call(
        _edge_embed_body,
        out_shape=jax.ShapeDtypeStruct((E, DN), jnp.float32),
        grid=(E // _EBLK,),
        in_specs=[
            pl.BlockSpec((_EBLK, DE), lambda i: (i, 0)),
            pl.BlockSpec((DE, DN), lambda i: (0, 0)),
            pl.BlockSpec((1, DN), lambda i: (0, 0)),
        ],
        out_specs=pl.BlockSpec((_EBLK, DN), lambda i: (i, 0)),
    )(edge_attr, W, b.reshape(1, DN))


_SC_MESH = plsc.VectorSubcoreMesh(core_axis_name="c", subcore_axis_name="s")


def _sc_aggregate(x, src3, dst3, e):
    @functools.partial(
        pl.kernel,
        out_type=jax.ShapeDtypeStruct((2, NPAD, DN), jnp.float32),
        mesh=_SC_MESH,
        scratch_types=[
            pltpu.VMEM((IB, CHUNK), jnp.int32),
            pltpu.VMEM((IB, CHUNK), jnp.int32),
            pltpu.VMEM((IB, CHUNK), jnp.int32),
            pltpu.VMEM((IB, CHUNK), jnp.int32),
            pltpu.VMEM((CHUNK, DN), jnp.float32),
            pltpu.VMEM((CHUNK, DN), jnp.float32),
            pltpu.VMEM((CHUNK, DN), jnp.float32),
            pltpu.VMEM((CHUNK, DN), jnp.float32),
            pltpu.VMEM((16, DN), jnp.float32),
            pltpu.VMEM_SHARED((NPAD, DN), jnp.float32),
            pltpu.SemaphoreType.DMA,
            pltpu.SemaphoreType.DMA,
            pltpu.SemaphoreType.DMA,
            pltpu.SemaphoreType.DMA,
        ],
    )
    def agg(x_hbm, src_hbm, dst_hbm, e_hbm, out_hbm, sv0, sv1, dv0, dv1,
            xv0, xv1, ev0, ev1, zv, acc, gsem0, gsem1, esem0, esem1):
        cid = lax.axis_index("c")
        sid = lax.axis_index("s")
        sv = (sv0, sv1)
        dv = (dv0, dv1)
        dbufs = ((xv0, ev0, gsem0, esem0), (xv1, ev1, gsem1, esem1))

        wid = cid * 16 + sid
        ebase = wid * EPW

        def refill(batch):
            p = batch % 2
            pltpu.sync_copy(src_hbm.at[wid, pl.ds(batch * IB, IB)], sv[p])
            pltpu.sync_copy(dst_hbm.at[wid, pl.ds(batch * IB, IB)], dv[p])

        def issue(b, p, row, eoff):
            xv, ev, gsem, esem = dbufs[b]
            pltpu.async_copy(x_hbm.at[sv[p].at[row]], xv, gsem)
            pltpu.async_copy(e_hbm.at[pl.ds(eoff, CHUNK)], ev, esem)

        def process(b, p, row, eoff, nxt):
            xv, ev, gsem, esem = dbufs[b]
            pltpu.make_async_copy(x_hbm.at[sv[p].at[row]], xv, gsem).wait()
            pltpu.make_async_copy(e_hbm.at[pl.ds(eoff, CHUNK)], ev, esem).wait()
            if nxt is not None:
                issue(*nxt)

            @pl.loop(0, CHUNK)
            def _row(r):
                for k in range(8):
                    sl = (r, pl.ds(k * 16, 16))
                    xv[sl] = jnp.maximum(xv[sl] + ev[sl], 0.0)

            pltpu.sync_copy(xv, acc.at[dv[p].at[row]], add=True)

        @pl.loop(0, 16)
        def _zfill(r):
            for j in range(8):
                zv[r, pl.ds(j * 16, 16)] = jnp.zeros((16,), jnp.float32)

        refill(0)

        @pl.loop(0, 40)
        def _zcopy(k):
            pltpu.sync_copy(zv, acc.at[pl.ds(sid * 640 + k * 16, 16)])

        plsc.subcore_barrier()

        issue(0, 0, 0, ebase)

        batch_sizes = [IB] * (NCHUNK // IB) + ([NCHUNK % IB] if NCHUNK % IB else [])
        nbatch = len(batch_sizes)
        for batch in range(nbatch):
            p = batch % 2
            ebb = ebase + batch * IB * CHUNK
            if batch + 1 < nbatch:
                refill(batch + 1)
            n = batch_sizes[batch]

            @pl.loop(0, n - 2, step=2)
            def _inner(i, p=p, ebb=ebb):
                process(0, p, i, ebb + i * CHUNK,
                        (1, p, i + 1, ebb + (i + 1) * CHUNK))
                process(1, p, i + 1, ebb + (i + 1) * CHUNK,
                        (0, p, i + 2, ebb + (i + 2) * CHUNK))

            if batch + 1 < nbatch:
                nxt_last = (0, 1 - p, 0, ebase + (batch + 1) * IB * CHUNK)
            else:
                nxt_last = None
            process(0, p, n - 2, ebb + (n - 2) * CHUNK,
                    (1, p, n - 1, ebb + (n - 1) * CHUNK))
            process(1, p, n - 1, ebb + (n - 1) * CHUNK, nxt_last)

        plsc.subcore_barrier()
        pltpu.sync_copy(
            acc.at[pl.ds(sid * 640, 640)],
            out_hbm.at[cid, pl.ds(sid * 640, 640)],
        )

    return agg(x, src3, dst3, e)


def _node_update_body(x_ref, p_ref, w1_ref, b1_ref, w2_ref, b2_ref, g_ref,
                      bb_ref, o_ref):
    h = x_ref[...] + p_ref[0, :N, :] + p_ref[1, :N, :]
    t = jnp.dot(h, w1_ref[...], preferred_element_type=jnp.float32) + b1_ref[...]
    t = jnp.maximum(t, 0.0)
    h2 = jnp.dot(t, w2_ref[...], preferred_element_type=jnp.float32) + b2_ref[...]
    mu = jnp.mean(h2, axis=0, keepdims=True)
    var = jnp.mean(jnp.square(h2 - mu), axis=0, keepdims=True)
    hb = g_ref[...] * (h2 - mu) * lax.rsqrt(var + 1e-5) + bb_ref[...]
    o_ref[...] = jnp.maximum(hb, 0.0)


def _node_update(x, parts, W1, b1, W2, b2, g, bb):
    return pl.pallas_call(
        _node_update_body,
        out_shape=jax.ShapeDtypeStruct((N, DN), jnp.float32),
    )(x, parts, W1, b1.reshape(1, -1), W2, b2.reshape(1, -1),
      g.reshape(1, -1), bb.reshape(1, -1))


def _final_mlp_body(h_ref, w0_ref, b0_ref, w1_ref, b1_ref, o_ref):
    t = jnp.dot(h_ref[...], w0_ref[...], preferred_element_type=jnp.float32)
    t = jnp.maximum(t + b0_ref[...], 0.0)
    o_ref[...] = (
        jnp.dot(t, w1_ref[...], preferred_element_type=jnp.float32) + b1_ref[...]
    )


def _final_mlp(h, Wm0, bm0, Wm1, bm1):
    return pl.pallas_call(
        _final_mlp_body,
        out_shape=jax.ShapeDtypeStruct((N, 1), jnp.float32),
    )(h, Wm0, bm0.reshape(1, -1), Wm1, bm1.reshape(1, -1))


def kernel(x, edge_index, edge_attr, We0, be0, We1, be1, W00, b00, W01, b01,
           W10, b10, W11, b11, g0, bb0, g1, bb1, Wm0, bm0, Wm1, bm1):
    idx3 = edge_index.reshape(2, NW, NCHUNK, CHUNK)
    idx3 = jnp.pad(idx3, ((0, 0), (0, 0), (0, NIBPAD - NCHUNK), (0, 0)))
    src = idx3[0]
    dst = idx3[1]

    e0 = _edge_embed(edge_attr, We0, be0)
    e1 = _edge_embed(edge_attr, We1, be1)

    p0 = _sc_aggregate(x, src, dst, e0)
    h = _node_update(x, p0, W00, b00, W01, b01, g0, bb0)

    p1 = _sc_aggregate(h, src, dst, e1)
    h = _node_update(h, p1, W10, b10, W11, b11, g1, bb1)

    return _final_mlp(h, Wm0, bm0, Wm1, bm1)


# speedup vs baseline: 4.4053x; 1.1434x over previous
import functools

import jax
import jax.numpy as jnp
from jax import lax
from jax.experimental import pallas as pl
from jax.experimental.pallas import tpu as pltpu
from jax.experimental.pallas import tpu_sc as plsc

N = 10000
E = 320000
DN = 128
DE = 16
NPAD = 10240
NW = 32
EPW = E // NW
CHUNK = 40
NCHUNK = EPW // CHUNK
IB = 32
NIBPAD = 256


_EBLK = 3200


def _edge_embed_body(ea_ref, w_ref, b_ref, e_ref):
    e_ref[...] = (
        jnp.dot(ea_ref[...], w_ref[...], preferred_element_type=jnp.float32)
        + b_ref[...]
    )


def _edge_embed(edge_attr, W, b):
    return pl.pallas_call(
        _edge_embed_body,
        out_shape=jax.ShapeDtypeStruct((E, DN), jnp.float32),
        grid=(E // _EBLK,),
        in_specs=[
            pl.BlockSpec((_EBLK, DE), lambda i: (i, 0)),
            pl.BlockSpec((DE, DN), lambda i: (0, 0)),
            pl.BlockSpec((1, DN), lambda i: (0, 0)),
        ],
        out_specs=pl.BlockSpec((_EBLK, DN), lambda i: (i, 0)),
    )(edge_attr, W, b.reshape(1, DN))


_SC_MESH = plsc.VectorSubcoreMesh(core_axis_name="c", subcore_axis_name="s")


def _sc_aggregate(x, src3, dst3, e):
    @functools.partial(
        pl.kernel,
        out_type=jax.ShapeDtypeStruct((2, NPAD, DN), jnp.float32),
        mesh=_SC_MESH,
        scratch_types=[
            pltpu.VMEM((IB, CHUNK), jnp.int32),
            pltpu.VMEM((IB, CHUNK), jnp.int32),
            pltpu.VMEM((IB, CHUNK), jnp.int32),
            pltpu.VMEM((IB, CHUNK), jnp.int32),
            pltpu.VMEM((CHUNK, DN), jnp.float32),
            pltpu.VMEM((CHUNK, DN), jnp.float32),
            pltpu.VMEM((CHUNK, DN), jnp.float32),
            pltpu.VMEM((CHUNK, DN), jnp.float32),
            pltpu.VMEM((CHUNK, DN), jnp.float32),
            pltpu.VMEM((CHUNK, DN), jnp.float32),
            pltpu.VMEM((8, DN), jnp.float32),
            pltpu.VMEM_SHARED((NPAD, DN), jnp.float32),
            pltpu.SemaphoreType.DMA,
            pltpu.SemaphoreType.DMA,
            pltpu.SemaphoreType.DMA,
            pltpu.SemaphoreType.DMA,
            pltpu.SemaphoreType.DMA,
            pltpu.SemaphoreType.DMA,
        ],
    )
    def agg(x_hbm, src_hbm, dst_hbm, e_hbm, out_hbm, sv0, sv1, dv0, dv1,
            xv0, xv1, ev0, ev1, mv0, mv1, zv, acc,
            gsem0, gsem1, esem0, esem1, ssem0, ssem1):
        cid = lax.axis_index("c")
        sid = lax.axis_index("s")
        sv = (sv0, sv1)
        dv = (dv0, dv1)
        gbufs = ((xv0, ev0, gsem0, esem0), (xv1, ev1, gsem1, esem1))
        sbufs = ((mv0, ssem0), (mv1, ssem1))

        wid = cid * 16 + sid
        ebase = wid * EPW

        def refill(batch):
            p = batch % 2
            pltpu.sync_copy(src_hbm.at[wid, pl.ds(batch * IB, IB)], sv[p])
            pltpu.sync_copy(dst_hbm.at[wid, pl.ds(batch * IB, IB)], dv[p])

        def issue(b, p, row, eoff):
            xv, ev, gsem, esem = gbufs[b]
            pltpu.async_copy(x_hbm.at[sv[p].at[row]], xv, gsem)
            pltpu.async_copy(e_hbm.at[pl.ds(eoff, CHUNK)], ev, esem)

        def process(b, p, row, eoff, nxt, wait_scatter=True):
            xv, ev, gsem, esem = gbufs[b]
            mv, ssem = sbufs[b]
            if nxt is not None:
                issue(*nxt)
            pltpu.make_async_copy(x_hbm.at[sv[p].at[row]], xv, gsem).wait()
            pltpu.make_async_copy(e_hbm.at[pl.ds(eoff, CHUNK)], ev, esem).wait()
            if wait_scatter:
                pltpu.make_async_copy(mv, acc.at[dv[p].at[row]], ssem).wait()

            @pl.loop(0, CHUNK)
            def _row(r):
                for k in range(8):
                    sl = (r, pl.ds(k * 16, 16))
                    mv[sl] = jnp.maximum(xv[sl] + ev[sl], 0.0)

            pltpu.async_copy(mv, acc.at[dv[p].at[row]], ssem, add=True)

        @pl.loop(0, 8)
        def _zfill(r):
            for j in range(8):
                zv[r, pl.ds(j * 16, 16)] = jnp.zeros((16,), jnp.float32)

        refill(0)

        @pl.loop(0, 80)
        def _zcopy(k):
            pltpu.sync_copy(zv, acc.at[pl.ds(sid * 640 + k * 8, 8)])

        plsc.subcore_barrier()

        issue(0, 0, 0, ebase)

        batch_sizes = [IB] * (NCHUNK // IB) + ([NCHUNK % IB] if NCHUNK % IB else [])
        nbatch = len(batch_sizes)
        for batch in range(nbatch):
            p = batch % 2
            ebb = ebase + batch * IB * CHUNK
            if batch + 1 < nbatch:
                refill(batch + 1)
            n = batch_sizes[batch]

            lo = 0
            if batch == 0:
                process(0, p, 0, ebb, (1, p, 1, ebb + CHUNK),
                        wait_scatter=False)
                process(1, p, 1, ebb + CHUNK, (0, p, 2, ebb + 2 * CHUNK),
                        wait_scatter=False)
                lo = 2

            @pl.loop(lo, n - 2, step=2)
            def _inner(i, p=p, ebb=ebb):
                process(0, p, i, ebb + i * CHUNK,
                        (1, p, i + 1, ebb + (i + 1) * CHUNK))
                process(1, p, i + 1, ebb + (i + 1) * CHUNK,
                        (0, p, i + 2, ebb + (i + 2) * CHUNK))

            if batch + 1 < nbatch:
                nxt_last = (0, 1 - p, 0, ebase + (batch + 1) * IB * CHUNK)
            else:
                nxt_last = None
            process(0, p, n - 2, ebb + (n - 2) * CHUNK,
                    (1, p, n - 1, ebb + (n - 1) * CHUNK))
            process(1, p, n - 1, ebb + (n - 1) * CHUNK, nxt_last)

        pf = (nbatch - 1) % 2
        nl = batch_sizes[-1]
        pltpu.make_async_copy(mv0, acc.at[dv[pf].at[nl - 2]], ssem0).wait()
        pltpu.make_async_copy(mv1, acc.at[dv[pf].at[nl - 1]], ssem1).wait()

        plsc.subcore_barrier()
        pltpu.sync_copy(
            acc.at[pl.ds(sid * 640, 640)],
            out_hbm.at[cid, pl.ds(sid * 640, 640)],
        )

    return agg(x, src3, dst3, e)


def _node_update_body(x_ref, p_ref, w1_ref, b1_ref, w2_ref, b2_ref, g_ref,
                      bb_ref, o_ref):
    h = x_ref[...] + p_ref[0, :N, :] + p_ref[1, :N, :]
    t = jnp.dot(h, w1_ref[...], preferred_element_type=jnp.float32) + b1_ref[...]
    t = jnp.maximum(t, 0.0)
    h2 = jnp.dot(t, w2_ref[...], preferred_element_type=jnp.float32) + b2_ref[...]
    mu = jnp.mean(h2, axis=0, keepdims=True)
    var = jnp.mean(jnp.square(h2 - mu), axis=0, keepdims=True)
    hb = g_ref[...] * (h2 - mu) * lax.rsqrt(var + 1e-5) + bb_ref[...]
    o_ref[...] = jnp.maximum(hb, 0.0)


def _node_update(x, parts, W1, b1, W2, b2, g, bb):
    return pl.pallas_call(
        _node_update_body,
        out_shape=jax.ShapeDtypeStruct((N, DN), jnp.float32),
    )(x, parts, W1, b1.reshape(1, -1), W2, b2.reshape(1, -1),
      g.reshape(1, -1), bb.reshape(1, -1))


def _final_mlp_body(h_ref, w0_ref, b0_ref, w1_ref, b1_ref, o_ref):
    t = jnp.dot(h_ref[...], w0_ref[...], preferred_element_type=jnp.float32)
    t = jnp.maximum(t + b0_ref[...], 0.0)
    o_ref[...] = (
        jnp.dot(t, w1_ref[...], preferred_element_type=jnp.float32) + b1_ref[...]
    )


def _final_mlp(h, Wm0, bm0, Wm1, bm1):
    return pl.pallas_call(
        _final_mlp_body,
        out_shape=jax.ShapeDtypeStruct((N, 1), jnp.float32),
    )(h, Wm0, bm0.reshape(1, -1), Wm1, bm1.reshape(1, -1))


def kernel(x, edge_index, edge_attr, We0, be0, We1, be1, W00, b00, W01, b01,
           W10, b10, W11, b11, g0, bb0, g1, bb1, Wm0, bm0, Wm1, bm1):
    idx3 = edge_index.reshape(2, NW, NCHUNK, CHUNK)
    idx3 = jnp.pad(idx3, ((0, 0), (0, 0), (0, NIBPAD - NCHUNK), (0, 0)))
    src = idx3[0]
    dst = idx3[1]

    e0 = _edge_embed(edge_attr, We0, be0)
    e1 = _edge_embed(edge_attr, We1, be1)

    p0 = _sc_aggregate(x, src, dst, e0)
    h = _node_update(x, p0, W00, b00, W01, b01, g0, bb0)

    p1 = _sc_aggregate(h, src, dst, e1)
    h = _node_update(h, p1, W10, b10, W11, b11, g1, bb1)

    return _final_mlp(h, Wm0, bm0, Wm1, bm1)
